# Initial kernel scaffold; baseline (speedup 1.0000x reference)
import jax
import jax.numpy as jnp
from jax.experimental import pallas as pl


def kernel(x, params):
    raise NotImplementedError("write your pallas kernel here")


# speedup vs baseline: 2.2980x; 2.2980x over previous
import functools
import math

import jax
import jax.numpy as jnp
from jax.experimental import pallas as pl

_F32 = jnp.float32
_DM = 512
_INV_SQRT_DM = 1.0 / math.sqrt(float(_DM))


def _full(shape):
    return pl.BlockSpec(shape, lambda *_: (0,) * len(shape))


def _embed_body(x_ref, w0_ref, b0_ref, w1_ref, b1_ref, o_ref):
    x = x_ref[0]
    h = jnp.maximum(
        jnp.dot(x, w0_ref[...], preferred_element_type=_F32) + b0_ref[...], 0.0)
    o_ref[0] = jnp.dot(h, w1_ref[...], preferred_element_type=_F32) + b1_ref[...]


def _embed(x, w0, b0, w1, b1):
    B, N, _ = x.shape
    C = w1.shape[1]
    return pl.pallas_call(
        _embed_body,
        grid=(B,),
        in_specs=[
            pl.BlockSpec((1, N, 3), lambda b: (b, 0, 0)),
            _full(w0.shape), _full(b0.shape), _full(w1.shape), _full(b1.shape),
        ],
        out_specs=pl.BlockSpec((1, N, C), lambda b: (b, 0, 0)),
        out_shape=jax.ShapeDtypeStruct((B, N, C), _F32),
    )(x, w0, b0, w1, b1)


def _proj_body(f_ref, w_ref, b_ref, wqkv_ref, o_ref):
    x = jnp.dot(f_ref[0], w_ref[...], preferred_element_type=_F32) + b_ref[...]
    o_ref[0] = jnp.dot(x, wqkv_ref[...], preferred_element_type=_F32)


def _proj(features, fc1_w, fc1_b, wqkv):
    B, N, d = features.shape
    return pl.pallas_call(
        _proj_body,
        grid=(B,),
        in_specs=[
            pl.BlockSpec((1, N, d), lambda b: (b, 0, 0)),
            _full(fc1_w.shape), _full(fc1_b.shape), _full(wqkv.shape),
        ],
        out_specs=pl.BlockSpec((1, N, 3 * _DM), lambda b: (b, 0, 0)),
        out_shape=jax.ShapeDtypeStruct((B, N, 3 * _DM), _F32),
    )(features, fc1_w, fc1_b, wqkv)


def _attn_body(xyz_ref, xyzT_ref, xyzt_ref, q_ref, kv_ref, pre_ref,
               d1_ref, d1b_ref, d2_ref, d2b_ref,
               g1_ref, g1b_ref, g2_ref, g2b_ref,
               fc2_ref, fc2b_ref, o_ref, *, N, TN, K_eff):
    xf = xyz_ref[0]
    xfT = xyzT_ref[0]
    xt = xyzt_ref[0]
    q = q_ref[0]
    kv = kv_ref[0]
    d1 = d1_ref[...]
    d1b = d1b_ref[...]
    d2 = d2_ref[...]
    d2b = d2b_ref[...]
    g1 = g1_ref[...]
    g1b = g1b_ref[...]
    g2 = g2_ref[...]
    g2b = g2b_ref[...]

    rn_t = jnp.sum(xt * xt, axis=1, keepdims=True)
    rn_f = jnp.sum(xfT * xfT, axis=0, keepdims=True)
    prod = jnp.dot(xt, xfT, preferred_element_type=_F32)
    dists = rn_t + rn_f - 2.0 * prod

    iota = jax.lax.broadcasted_iota(jnp.int32, (TN, N), 1)

    m = None
    denom = None
    acc = None
    for j in range(K_eff):
        mn = jnp.min(dists, axis=1, keepdims=True)
        idx = jnp.min(jnp.where(dists == mn, iota, N), axis=1, keepdims=True)
        sel = iota == idx
        oh = sel.astype(_F32)
        dists = jnp.where(sel, jnp.inf, dists)

        kvj = jnp.dot(oh, kv, preferred_element_type=_F32,
                      precision=jax.lax.Precision.HIGHEST)
        kkj = kvj[:, :_DM]
        vj = kvj[:, _DM:]
        xyzj = jnp.dot(oh, xf, preferred_element_type=_F32,
                       precision=jax.lax.Precision.HIGHEST)
        pos = xt - xyzj
        pe = jnp.dot(
            jnp.maximum(jnp.dot(pos, d1, preferred_element_type=_F32) + d1b, 0.0),
            d2, preferred_element_type=_F32) + d2b
        u = q - kkj + pe
        a = jnp.dot(
            jnp.maximum(jnp.dot(u, g1, preferred_element_type=_F32) + g1b, 0.0),
            g2, preferred_element_type=_F32) + g2b
        z = a * _INV_SQRT_DM
        w = vj + pe
        if j == 0:
            m = z
            denom = jnp.ones_like(z)
            acc = w
        else:
            m2 = jnp.maximum(m, z)
            s_old = jnp.exp(m - m2)
            s_new = jnp.exp(z - m2)
            denom = denom * s_old + s_new
            acc = acc * s_old + s_new * w
            m = m2

    res = acc / denom
    o_ref[0] = (jnp.dot(res, fc2_ref[...], preferred_element_type=_F32)
                + fc2b_ref[...] + pre_ref[0])


def _attn_block(xyz, xyzT, q, kv, pre, p, pr):
    B, N, _ = xyz.shape
    d = pre.shape[-1]
    TN = min(N, 128)
    nt = N // TN
    K_eff = min(16, N)
    body = functools.partial(_attn_body, N=N, TN=TN, K_eff=K_eff)
    return pl.pallas_call(
        body,
        grid=(B, nt),
        in_specs=[
            pl.BlockSpec((1, N, 3), lambda b, t: (b, 0, 0)),
            pl.BlockSpec((1, 3, N), lambda b, t: (b, 0, 0)),
            pl.BlockSpec((1, TN, 3), lambda b, t: (b, t, 0)),
            pl.BlockSpec((1, TN, _DM), lambda b, t: (b, t, 0)),
            pl.BlockSpec((1, N, 2 * _DM), lambda b, t: (b, 0, 0)),
            pl.BlockSpec((1, TN, d), lambda b, t: (b, t, 0)),
            _full(p['d1_w'].shape), _full(pr['d1_b'].shape),
            _full(p['d2_w'].shape), _full(pr['d2_b'].shape),
            _full(p['g1_w'].shape), _full(pr['g1_b'].shape),
            _full(p['g2_w'].shape), _full(pr['g2_b'].shape),
            _full(p['fc2_w'].shape), _full(pr['fc2_b'].shape),
        ],
        out_specs=pl.BlockSpec((1, TN, d), lambda b, t: (b, t, 0)),
        out_shape=jax.ShapeDtypeStruct((B, N, d), _F32),
    )(xyz, xyzT, xyz, q, kv, pre,
      p['d1_w'], pr['d1_b'], p['d2_w'], pr['d2_b'],
      p['g1_w'], pr['g1_b'], p['g2_w'], pr['g2_b'],
      p['fc2_w'], pr['fc2_b'])


def _fps_body(xT_ref, o_ref, *, N, npoint):
    xr = xT_ref[0, 0:1, :]
    yr = xT_ref[0, 1:2, :]
    zr = xT_ref[0, 2:3, :]
    ln = jax.lax.broadcasted_iota(jnp.int32, (1, N), 1)
    lnp = jax.lax.broadcasted_iota(jnp.int32, (1, npoint), 1)

    def step(i, carry):
        dist, far, idxs = carry
        idxs = jnp.where(lnp == i, far, idxs)
        sel = ln == far
        cx = jnp.sum(jnp.where(sel, xr, 0.0))
        cy = jnp.sum(jnp.where(sel, yr, 0.0))
        cz = jnp.sum(jnp.where(sel, zr, 0.0))
        d = (xr - cx) ** 2 + (yr - cy) ** 2 + (zr - cz) ** 2
        dist = jnp.minimum(dist, d)
        mx = jnp.max(dist)
        far = jnp.min(jnp.where(dist == mx, ln, N)).astype(jnp.int32)
        return dist, far, idxs

    dist0 = jnp.full((1, N), 1e10, _F32)
    init = (dist0, jnp.int32(0), jnp.zeros((1, npoint), jnp.int32))
    _, _, idxs = jax.lax.fori_loop(0, npoint, step, init)
    o_ref[0] = idxs


def _fps(xyzT, npoint):
    B, _, N = xyzT.shape
    body = functools.partial(_fps_body, N=N, npoint=npoint)
    return pl.pallas_call(
        body,
        grid=(B,),
        in_specs=[pl.BlockSpec((1, 3, N), lambda b: (b, 0, 0))],
        out_specs=pl.BlockSpec((1, 1, npoint), lambda b: (b, 0, 0)),
        out_shape=jax.ShapeDtypeStruct((B, 1, npoint), jnp.int32),
    )(xyzT)


def _sa_body(xyz_ref, xyzT_ref, pts_ref, fi_ref,
             w1a_ref, w1b_ref, b1_ref, s1g_ref, s1b_ref, s1m_ref, s1v_ref,
             w2_ref, b2_ref, s2g_ref, s2b_ref, s2m_ref, s2v_ref,
             oxyz_ref, opts_ref, *, N, npoint, K_eff):
    xf = xyz_ref[0]
    xfT = xyzT_ref[0]
    pts = pts_ref[0]
    fi = fi_ref[0]

    iota = jax.lax.broadcasted_iota(jnp.int32, (npoint, N), 1)
    ohf = (iota == fi).astype(_F32)
    new_xyz = jnp.dot(ohf, xf, preferred_element_type=_F32,
                      precision=jax.lax.Precision.HIGHEST)
    oxyz_ref[0] = new_xyz

    rn_n = jnp.sum(new_xyz * new_xyz, axis=1, keepdims=True)
    rn_f = jnp.sum(xfT * xfT, axis=0, keepdims=True)
    dists = rn_n + rn_f - 2.0 * jnp.dot(new_xyz, xfT, preferred_element_type=_F32)

    scale1 = s1g_ref[...] * jax.lax.rsqrt(s1v_ref[...] + 1e-5)
    scale2 = s2g_ref[...] * jax.lax.rsqrt(s2v_ref[...] + 1e-5)
    mean1 = s1m_ref[...]
    beta1 = s1b_ref[...]
    mean2 = s2m_ref[...]
    beta2 = s2b_ref[...]
    w1a = w1a_ref[...]
    w1b = w1b_ref[...]
    b1 = b1_ref[...]
    w2 = w2_ref[...]
    b2 = b2_ref[...]

    best = None
    for j in range(K_eff):
        mn = jnp.min(dists, axis=1, keepdims=True)
        idx = jnp.min(jnp.where(dists == mn, iota, N), axis=1, keepdims=True)
        sel = iota == idx
        oh = sel.astype(_F32)
        dists = jnp.where(sel, jnp.inf, dists)

        gx = jnp.dot(oh, xf, preferred_element_type=_F32,
                     precision=jax.lax.Precision.HIGHEST) - new_xyz
        gp = jnp.dot(oh, pts, preferred_element_type=_F32,
                     precision=jax.lax.Precision.HIGHEST)
        h1 = (jnp.dot(gx, w1a, preferred_element_type=_F32)
              + jnp.dot(gp, w1b, preferred_element_type=_F32) + b1)
        h1 = jnp.maximum((h1 - mean1) * scale1 + beta1, 0.0)
        h2 = jnp.dot(h1, w2, preferred_element_type=_F32) + b2
        h2 = jnp.maximum((h2 - mean2) * scale2 + beta2, 0.0)
        best = h2 if best is None else jnp.maximum(best, h2)

    opts_ref[0] = best


def _sa(xyz, xyzT, points, fi_col, p, pr):
    B, N, _ = xyz.shape
    npoint = fi_col.shape[1]
    Cin = points.shape[-1]
    Cout = p['conv1_w'].shape[1]
    K_eff = min(16, N)
    w1a = p['conv1_w'][:3]
    w1b = p['conv1_w'][3:]
    body = functools.partial(_sa_body, N=N, npoint=npoint, K_eff=K_eff)
    return pl.pallas_call(
        body,
        grid=(B,),
        in_specs=[
            pl.BlockSpec((1, N, 3), lambda b: (b, 0, 0)),
            pl.BlockSpec((1, 3, N), lambda b: (b, 0, 0)),
            pl.BlockSpec((1, N, Cin), lambda b: (b, 0, 0)),
            pl.BlockSpec((1, npoint, 1), lambda b: (b, 0, 0)),
            _full(w1a.shape), _full(w1b.shape), _full(pr['conv1_b'].shape),
            _full(pr['bn1_gamma'].shape), _full(pr['bn1_beta'].shape),
            _full(pr['bn1_mean'].shape), _full(pr['bn1_var'].shape),
            _full(p['conv2_w'].shape), _full(pr['conv2_b'].shape),
            _full(pr['bn2_gamma'].shape), _full(pr['bn2_beta'].shape),
            _full(pr['bn2_mean'].shape), _full(pr['bn2_var'].shape),
        ],
        out_specs=[
            pl.BlockSpec((1, npoint, 3), lambda b: (b, 0, 0)),
            pl.BlockSpec((1, npoint, Cout), lambda b: (b, 0, 0)),
        ],
        out_shape=[
            jax.ShapeDtypeStruct((B, npoint, 3), _F32),
            jax.ShapeDtypeStruct((B, npoint, Cout), _F32),
        ],
    )(xyz, xyzT, points, fi_col,
      w1a, w1b, pr['conv1_b'],
      pr['bn1_gamma'], pr['bn1_beta'], pr['bn1_mean'], pr['bn1_var'],
      p['conv2_w'], pr['conv2_b'],
      pr['bn2_gamma'], pr['bn2_beta'], pr['bn2_mean'], pr['bn2_var'])


def _row(v):
    return v.reshape(1, -1)


def kernel(x, params):
    p = params
    B, N0, _ = x.shape
    xyz = x[..., :3]

    h = _embed(x, p['fc1_0_w'], _row(p['fc1_0_b']),
               p['fc1_1_w'], _row(p['fc1_1_b']))

    def tf_block(pb, xyz_, feats_):
        pr = {k: _row(pb[k]) for k in
              ('fc1_b', 'fc2_b', 'd1_b', 'd2_b', 'g1_b', 'g2_b')}
        xyzT = jnp.transpose(xyz_, (0, 2, 1))
        wqkv = jnp.concatenate([pb['wq'], pb['wk'], pb['wv']], axis=1)
        qkv = _proj(feats_, pb['fc1_w'], pr['fc1_b'], wqkv)
        q = qkv[..., :_DM]
        kv = qkv[..., _DM:]
        return _attn_block(xyz_, xyzT, q, kv, feats_, pb, pr)

    points = tf_block(p['tf0'], xyz, h)
    feats = [(xyz, points)]
    cur = xyz
    for i in range(4):
        npoint = N0 // (4 ** (i + 1))
        pb = p['td%d' % i]
        pr = {k: _row(pb[k]) for k in
              ('conv1_b', 'conv2_b', 'bn1_gamma', 'bn1_beta', 'bn1_mean',
               'bn1_var', 'bn2_gamma', 'bn2_beta', 'bn2_mean', 'bn2_var')}
        xyzT = jnp.transpose(cur, (0, 2, 1))
        fi = _fps(xyzT, npoint)
        fi_col = fi.reshape(B, npoint, 1)
        cur, points = _sa(cur, xyzT, points, fi_col, pb, pr)
        points = tf_block(p['tf%d' % (i + 1)], cur, points)
        feats.append((cur, points))

    outs = [points]
    for xz, f in feats:
        outs.append(xz)
        outs.append(f)
    return tuple(outs)


# speedup vs baseline: 3.3889x; 1.4747x over previous
import functools
import math

import jax
import jax.numpy as jnp
from jax.experimental import pallas as pl

_F32 = jnp.float32
_DM = 512
_INV_SQRT_DM = 1.0 / math.sqrt(float(_DM))


def _full(shape):
    return pl.BlockSpec(shape, lambda *_: (0,) * len(shape))


def _embed_body(x_ref, w0_ref, b0_ref, w1_ref, b1_ref, o_ref):
    x = x_ref[0]
    h = jnp.maximum(
        jnp.dot(x, w0_ref[...], preferred_element_type=_F32) + b0_ref[...], 0.0)
    o_ref[0] = jnp.dot(h, w1_ref[...], preferred_element_type=_F32) + b1_ref[...]


def _embed(x, w0, b0, w1, b1):
    B, N, _ = x.shape
    C = w1.shape[1]
    return pl.pallas_call(
        _embed_body,
        grid=(B,),
        in_specs=[
            pl.BlockSpec((1, N, 3), lambda b: (b, 0, 0)),
            _full(w0.shape), _full(b0.shape), _full(w1.shape), _full(b1.shape),
        ],
        out_specs=pl.BlockSpec((1, N, C), lambda b: (b, 0, 0)),
        out_shape=jax.ShapeDtypeStruct((B, N, C), _F32),
    )(x, w0, b0, w1, b1)


def _proj_body(f_ref, w_ref, b_ref, wqkv_ref, q_ref, hi_ref, lo_ref):
    x = jnp.dot(f_ref[0], w_ref[...], preferred_element_type=_F32) + b_ref[...]
    qkv = jnp.dot(x, wqkv_ref[...], preferred_element_type=_F32)
    q_ref[0] = qkv[:, :_DM]
    kv = qkv[:, _DM:]
    hi = kv.astype(jnp.bfloat16)
    hi_ref[0] = hi
    lo_ref[0] = (kv - hi.astype(_F32)).astype(jnp.bfloat16)


def _proj(features, fc1_w, fc1_b, wqkv):
    B, N, d = features.shape
    return pl.pallas_call(
        _proj_body,
        grid=(B,),
        in_specs=[
            pl.BlockSpec((1, N, d), lambda b: (b, 0, 0)),
            _full(fc1_w.shape), _full(fc1_b.shape), _full(wqkv.shape),
        ],
        out_specs=[
            pl.BlockSpec((1, N, _DM), lambda b: (b, 0, 0)),
            pl.BlockSpec((1, N, 2 * _DM), lambda b: (b, 0, 0)),
            pl.BlockSpec((1, N, 2 * _DM), lambda b: (b, 0, 0)),
        ],
        out_shape=[
            jax.ShapeDtypeStruct((B, N, _DM), _F32),
            jax.ShapeDtypeStruct((B, N, 2 * _DM), jnp.bfloat16),
            jax.ShapeDtypeStruct((B, N, 2 * _DM), jnp.bfloat16),
        ],
    )(features, fc1_w, fc1_b, wqkv)


def _attn_body(xyz_ref, xyzT_ref, xyzt_ref, q_ref, kvhi_ref, kvlo_ref, pre_ref,
               d1_ref, d1b_ref, d2_ref, d2b_ref,
               g1_ref, g1b_ref, g2_ref, g2b_ref,
               fc2_ref, fc2b_ref, o_ref, *, N, TN, K_eff):
    xf = xyz_ref[0]
    xfT = xyzT_ref[0]
    xt = xyzt_ref[0]
    q = q_ref[0]
    kvhi = kvhi_ref[0]
    kvlo = kvlo_ref[0]
    d1 = d1_ref[...]
    d1b = d1b_ref[...]
    d2 = d2_ref[...]
    d2b = d2b_ref[...]
    g1 = g1_ref[...]
    g1b = g1b_ref[...]
    g2 = g2_ref[...]
    g2b = g2b_ref[...]

    rn_t = jnp.sum(xt * xt, axis=1, keepdims=True)
    rn_f = jnp.sum(xfT * xfT, axis=0, keepdims=True)
    prod = jnp.dot(xt, xfT, preferred_element_type=_F32)
    dists = rn_t + rn_f - 2.0 * prod

    iota = jax.lax.broadcasted_iota(jnp.int32, (TN, N), 1)

    m = None
    denom = None
    acc = None
    for j in range(K_eff):
        mn = jnp.min(dists, axis=1, keepdims=True)
        idx = jnp.min(jnp.where(dists == mn, iota, N), axis=1, keepdims=True)
        sel = iota == idx
        oh = sel.astype(_F32)
        dists = jnp.where(sel, jnp.inf, dists)

        ohb = oh.astype(jnp.bfloat16)
        kvj = (jnp.dot(ohb, kvhi, preferred_element_type=_F32)
               + jnp.dot(ohb, kvlo, preferred_element_type=_F32))
        kkj = kvj[:, :_DM]
        vj = kvj[:, _DM:]
        xyzj = jnp.dot(oh, xf, preferred_element_type=_F32,
                       precision=jax.lax.Precision.HIGHEST)
        pos = xt - xyzj
        pe = jnp.dot(
            jnp.maximum(jnp.dot(pos, d1, preferred_element_type=_F32) + d1b, 0.0),
            d2, preferred_element_type=_F32) + d2b
        u = q - kkj + pe
        a = jnp.dot(
            jnp.maximum(jnp.dot(u, g1, preferred_element_type=_F32) + g1b, 0.0),
            g2, preferred_element_type=_F32) + g2b
        z = a * _INV_SQRT_DM
        w = vj + pe
        if j == 0:
            m = z
            denom = jnp.ones_like(z)
            acc = w
        else:
            m2 = jnp.maximum(m, z)
            s_old = jnp.exp(m - m2)
            s_new = jnp.exp(z - m2)
            denom = denom * s_old + s_new
            acc = acc * s_old + s_new * w
            m = m2

    res = acc / denom
    o_ref[0] = (jnp.dot(res, fc2_ref[...], preferred_element_type=_F32)
                + fc2b_ref[...] + pre_ref[0])


def _attn_block(xyz, xyzT, q, kvhi, kvlo, pre, p, pr):
    B, N, _ = xyz.shape
    d = pre.shape[-1]
    TN = min(N, 128)
    nt = N // TN
    K_eff = min(16, N)
    body = functools.partial(_attn_body, N=N, TN=TN, K_eff=K_eff)
    return pl.pallas_call(
        body,
        grid=(B, nt),
        in_specs=[
            pl.BlockSpec((1, N, 3), lambda b, t: (b, 0, 0)),
            pl.BlockSpec((1, 3, N), lambda b, t: (b, 0, 0)),
            pl.BlockSpec((1, TN, 3), lambda b, t: (b, t, 0)),
            pl.BlockSpec((1, TN, _DM), lambda b, t: (b, t, 0)),
            pl.BlockSpec((1, N, 2 * _DM), lambda b, t: (b, 0, 0)),
            pl.BlockSpec((1, N, 2 * _DM), lambda b, t: (b, 0, 0)),
            pl.BlockSpec((1, TN, d), lambda b, t: (b, t, 0)),
            _full(p['d1_w'].shape), _full(pr['d1_b'].shape),
            _full(p['d2_w'].shape), _full(pr['d2_b'].shape),
            _full(p['g1_w'].shape), _full(pr['g1_b'].shape),
            _full(p['g2_w'].shape), _full(pr['g2_b'].shape),
            _full(p['fc2_w'].shape), _full(pr['fc2_b'].shape),
        ],
        out_specs=pl.BlockSpec((1, TN, d), lambda b, t: (b, t, 0)),
        out_shape=jax.ShapeDtypeStruct((B, N, d), _F32),
    )(xyz, xyzT, xyz, q, kvhi, kvlo, pre,
      p['d1_w'], pr['d1_b'], p['d2_w'], pr['d2_b'],
      p['g1_w'], pr['g1_b'], p['g2_w'], pr['g2_b'],
      p['fc2_w'], pr['fc2_b'])


def _fps_body(xT_ref, o_ref, *, N, npoint):
    xr = xT_ref[0, 0:1, :]
    yr = xT_ref[0, 1:2, :]
    zr = xT_ref[0, 2:3, :]
    ln = jax.lax.broadcasted_iota(jnp.int32, (1, N), 1)
    lnp = jax.lax.broadcasted_iota(jnp.int32, (1, npoint), 1)

    def step(i, carry):
        dist, far, idxs = carry
        idxs = jnp.where(lnp == i, far, idxs)
        sel = ln == far
        cx = jnp.sum(jnp.where(sel, xr, 0.0))
        cy = jnp.sum(jnp.where(sel, yr, 0.0))
        cz = jnp.sum(jnp.where(sel, zr, 0.0))
        d = (xr - cx) ** 2 + (yr - cy) ** 2 + (zr - cz) ** 2
        dist = jnp.minimum(dist, d)
        mx = jnp.max(dist)
        far = jnp.min(jnp.where(dist == mx, ln, N)).astype(jnp.int32)
        return dist, far, idxs

    dist0 = jnp.full((1, N), 1e10, _F32)
    init = (dist0, jnp.int32(0), jnp.zeros((1, npoint), jnp.int32))
    _, _, idxs = jax.lax.fori_loop(0, npoint, step, init)
    o_ref[0] = idxs


def _fps(xyzT, npoint):
    B, _, N = xyzT.shape
    body = functools.partial(_fps_body, N=N, npoint=npoint)
    return pl.pallas_call(
        body,
        grid=(B,),
        in_specs=[pl.BlockSpec((1, 3, N), lambda b: (b, 0, 0))],
        out_specs=pl.BlockSpec((1, 1, npoint), lambda b: (b, 0, 0)),
        out_shape=jax.ShapeDtypeStruct((B, 1, npoint), jnp.int32),
    )(xyzT)


def _sa_body(xyz_ref, xyzT_ref, pts_ref, fi_ref,
             w1a_ref, w1b_ref, b1_ref, s1g_ref, s1b_ref, s1m_ref, s1v_ref,
             w2_ref, b2_ref, s2g_ref, s2b_ref, s2m_ref, s2v_ref,
             oxyz_ref, opts_ref, *, N, npoint, K_eff):
    xf = xyz_ref[0]
    xfT = xyzT_ref[0]
    pts = pts_ref[0]
    fi = fi_ref[0]

    iota = jax.lax.broadcasted_iota(jnp.int32, (npoint, N), 1)
    ohf = (iota == fi).astype(_F32)
    new_xyz = jnp.dot(ohf, xf, preferred_element_type=_F32,
                      precision=jax.lax.Precision.HIGHEST)
    oxyz_ref[0] = new_xyz

    rn_n = jnp.sum(new_xyz * new_xyz, axis=1, keepdims=True)
    rn_f = jnp.sum(xfT * xfT, axis=0, keepdims=True)
    dists = rn_n + rn_f - 2.0 * jnp.dot(new_xyz, xfT, preferred_element_type=_F32)

    scale1 = s1g_ref[...] * jax.lax.rsqrt(s1v_ref[...] + 1e-5)
    scale2 = s2g_ref[...] * jax.lax.rsqrt(s2v_ref[...] + 1e-5)
    mean1 = s1m_ref[...]
    beta1 = s1b_ref[...]
    mean2 = s2m_ref[...]
    beta2 = s2b_ref[...]
    w1a = w1a_ref[...]
    w1b = w1b_ref[...]
    b1 = b1_ref[...]
    w2 = w2_ref[...]
    b2 = b2_ref[...]

    best = None
    for j in range(K_eff):
        mn = jnp.min(dists, axis=1, keepdims=True)
        idx = jnp.min(jnp.where(dists == mn, iota, N), axis=1, keepdims=True)
        sel = iota == idx
        oh = sel.astype(_F32)
        dists = jnp.where(sel, jnp.inf, dists)

        gx = jnp.dot(oh, xf, preferred_element_type=_F32,
                     precision=jax.lax.Precision.HIGHEST) - new_xyz
        gp = jnp.dot(oh, pts, preferred_element_type=_F32,
                     precision=jax.lax.Precision.HIGHEST)
        h1 = (jnp.dot(gx, w1a, preferred_element_type=_F32)
              + jnp.dot(gp, w1b, preferred_element_type=_F32) + b1)
        h1 = jnp.maximum((h1 - mean1) * scale1 + beta1, 0.0)
        h2 = jnp.dot(h1, w2, preferred_element_type=_F32) + b2
        h2 = jnp.maximum((h2 - mean2) * scale2 + beta2, 0.0)
        best = h2 if best is None else jnp.maximum(best, h2)

    opts_ref[0] = best


def _sa(xyz, xyzT, points, fi_col, p, pr):
    B, N, _ = xyz.shape
    npoint = fi_col.shape[1]
    Cin = points.shape[-1]
    Cout = p['conv1_w'].shape[1]
    K_eff = min(16, N)
    w1a = p['conv1_w'][:3]
    w1b = p['conv1_w'][3:]
    body = functools.partial(_sa_body, N=N, npoint=npoint, K_eff=K_eff)
    return pl.pallas_call(
        body,
        grid=(B,),
        in_specs=[
            pl.BlockSpec((1, N, 3), lambda b: (b, 0, 0)),
            pl.BlockSpec((1, 3, N), lambda b: (b, 0, 0)),
            pl.BlockSpec((1, N, Cin), lambda b: (b, 0, 0)),
            pl.BlockSpec((1, npoint, 1), lambda b: (b, 0, 0)),
            _full(w1a.shape), _full(w1b.shape), _full(pr['conv1_b'].shape),
            _full(pr['bn1_gamma'].shape), _full(pr['bn1_beta'].shape),
            _full(pr['bn1_mean'].shape), _full(pr['bn1_var'].shape),
            _full(p['conv2_w'].shape), _full(pr['conv2_b'].shape),
            _full(pr['bn2_gamma'].shape), _full(pr['bn2_beta'].shape),
            _full(pr['bn2_mean'].shape), _full(pr['bn2_var'].shape),
        ],
        out_specs=[
            pl.BlockSpec((1, npoint, 3), lambda b: (b, 0, 0)),
            pl.BlockSpec((1, npoint, Cout), lambda b: (b, 0, 0)),
        ],
        out_shape=[
            jax.ShapeDtypeStruct((B, npoint, 3), _F32),
            jax.ShapeDtypeStruct((B, npoint, Cout), _F32),
        ],
    )(xyz, xyzT, points, fi_col,
      w1a, w1b, pr['conv1_b'],
      pr['bn1_gamma'], pr['bn1_beta'], pr['bn1_mean'], pr['bn1_var'],
      p['conv2_w'], pr['conv2_b'],
      pr['bn2_gamma'], pr['bn2_beta'], pr['bn2_mean'], pr['bn2_var'])


def _row(v):
    return v.reshape(1, -1)


def kernel(x, params):
    p = params
    B, N0, _ = x.shape
    xyz = x[..., :3]

    h = _embed(x, p['fc1_0_w'], _row(p['fc1_0_b']),
               p['fc1_1_w'], _row(p['fc1_1_b']))

    def tf_block(pb, xyz_, feats_):
        pr = {k: _row(pb[k]) for k in
              ('fc1_b', 'fc2_b', 'd1_b', 'd2_b', 'g1_b', 'g2_b')}
        xyzT = jnp.transpose(xyz_, (0, 2, 1))
        wqkv = jnp.concatenate([pb['wq'], pb['wk'], pb['wv']], axis=1)
        q, kvhi, kvlo = _proj(feats_, pb['fc1_w'], pr['fc1_b'], wqkv)
        return _attn_block(xyz_, xyzT, q, kvhi, kvlo, feats_, pb, pr)

    points = tf_block(p['tf0'], xyz, h)
    feats = [(xyz, points)]
    cur = xyz
    for i in range(4):
        npoint = N0 // (4 ** (i + 1))
        pb = p['td%d' % i]
        pr = {k: _row(pb[k]) for k in
              ('conv1_b', 'conv2_b', 'bn1_gamma', 'bn1_beta', 'bn1_mean',
               'bn1_var', 'bn2_gamma', 'bn2_beta', 'bn2_mean', 'bn2_var')}
        xyzT = jnp.transpose(cur, (0, 2, 1))
        fi = _fps(xyzT, npoint)
        fi_col = fi.reshape(B, npoint, 1)
        cur, points = _sa(cur, xyzT, points, fi_col, pb, pr)
        points = tf_block(p['tf%d' % (i + 1)], cur, points)
        feats.append((cur, points))

    outs = [points]
    for xz, f in feats:
        outs.append(xz)
        outs.append(f)
    return tuple(outs)
